# baseline (speedup 1.0000x reference)
import functools

import jax
import jax.numpy as jnp
from jax import lax
from jax.experimental import pallas as pl
from jax.experimental.pallas import tpu as pltpu
from jax.experimental.pallas import tpu_sc as plsc

_N = 100000
_NSEG = 128
_NW = 16
_CH = 6272
_LAST_CH = _N - (_NW - 1) * _CH
_BINS = 144
_ROW = 384


def _make_sc_call():
    mesh = plsc.VectorSubcoreMesh(
        core_axis_name="c", subcore_axis_name="s", num_cores=1)

    @functools.partial(
        pl.kernel,
        mesh=mesh,
        out_type=jax.ShapeDtypeStruct((_NW * _ROW,), jnp.float32),
        compiler_params=pltpu.CompilerParams(needs_layout_passes=False),
        scratch_types=[
            pltpu.VMEM((_CH,), jnp.float32),
            pltpu.VMEM((_CH,), jnp.float32),
            pltpu.VMEM((_CH,), jnp.int32),
            pltpu.VMEM((_ROW,), jnp.float32),
            pltpu.SemaphoreType.DMA,
            pltpu.SemaphoreType.DMA,
            pltpu.SemaphoreType.DMA,
        ],
    )
    def sc_bins(pred_hbm, targ_hbm, batch_hbm, out_hbm,
                pred_v, targ_v, batch_v, bins_v, sem1, sem2, sem3):
        w = lax.axis_index("s")
        base = w * _CH

        zeros16 = jnp.zeros((16,), jnp.float32)
        ones16 = jnp.ones((16,), jnp.float32)
        lane = lax.iota(jnp.int32, 16)

        def stage(n):
            dst = pl.ds(0, n)
            c1 = pltpu.async_copy(pred_hbm.at[pl.ds(base, n)], pred_v.at[dst], sem1)
            c2 = pltpu.async_copy(targ_hbm.at[pl.ds(base, n)], targ_v.at[dst], sem2)
            c3 = pltpu.async_copy(batch_hbm.at[pl.ds(base, n)], batch_v.at[dst], sem3)
            c1.wait()
            c2.wait()
            c3.wait()

        @pl.when(w != _NW - 1)
        def _stage_full():
            stage(_CH)

        @pl.when(w == _NW - 1)
        def _stage_tail():
            stage(_LAST_CH)

        for j in range(_ROW // 16):
            bins_v[pl.ds(j * 16, 16)] = zeros16

        def accumulate(n_vregs):
            lane_base = lane * n_vregs

            def body(i, carry):
                idx = lane_base + i
                p = plsc.load_gather(pred_v, [idx])
                t = plsc.load_gather(targ_v, [idx])
                b = plsc.load_gather(batch_v, [idx])
                v = jnp.abs(p * p - t * t)
                plsc.addupdate_scatter(bins_v, [b], v)
                plsc.addupdate_scatter(bins_v, [b + _BINS], ones16)
                return carry
            lax.fori_loop(0, n_vregs, body, 0, unroll=4)

        @pl.when(w != _NW - 1)
        def _accum_full():
            accumulate(_CH // 16)

        @pl.when(w == _NW - 1)
        def _accum_tail():
            accumulate(_LAST_CH // 16)
            last_vec = batch_v[pl.ds(_LAST_CH - 16, 16)]
            bins_v[pl.ds(2 * _BINS, 16)] = (
                zeros16 + last_vec[15].astype(jnp.float32))

        pltpu.sync_copy(bins_v, out_hbm.at[pl.ds(w * _ROW, _ROW)])

    return sc_bins


_sc_call = _make_sc_call()


def _tc_combine_body(x_ref, o_ref):
    x = x_ref[...]
    sums = jnp.sum(x[:, 0:_NSEG], axis=0, keepdims=True)
    cnts = jnp.sum(x[:, _BINS:_BINS + _NSEG], axis=0, keepdims=True)
    max_b = x[_NW - 1, 2 * _BINS].astype(jnp.int32)
    losses = sums / cnts
    seg = lax.broadcasted_iota(jnp.int32, (1, _NSEG), 1)
    valid = seg <= max_b
    total = jnp.sum(jnp.where(valid, losses, jnp.zeros_like(losses)))
    n_graphs = (max_b + 1).astype(jnp.float32)
    o_ref[...] = jnp.full((1, 1), (total / n_graphs) * 10000.0, jnp.float32)


_tc_combine = pl.pallas_call(
    _tc_combine_body,
    out_shape=jax.ShapeDtypeStruct((1, 1), jnp.float32),
)


@jax.jit
def kernel(pred, target, batch, x):
    del x
    rows = _sc_call(pred, target, batch).reshape(_NW, _ROW)
    return _tc_combine(rows)[0, 0]


# speedup vs baseline: 1.0533x; 1.0533x over previous
import functools

import jax
import jax.numpy as jnp
from jax import lax
from jax.experimental import pallas as pl
from jax.experimental.pallas import tpu as pltpu
from jax.experimental.pallas import tpu_sc as plsc

_N = 100000
_NSEG = 128
_NW = 32
_CH = 3136
_LAST_CH = _N - (_NW - 1) * _CH
_BINS = 144
_ROW = 384


def _make_sc_call():
    mesh = plsc.VectorSubcoreMesh(
        core_axis_name="c", subcore_axis_name="s", num_cores=2)

    @functools.partial(
        pl.kernel,
        mesh=mesh,
        out_type=jax.ShapeDtypeStruct((_NW * _ROW,), jnp.float32),
        compiler_params=pltpu.CompilerParams(needs_layout_passes=False),
        scratch_types=[
            pltpu.VMEM((_CH,), jnp.float32),
            pltpu.VMEM((_CH,), jnp.float32),
            pltpu.VMEM((_CH,), jnp.int32),
            pltpu.VMEM((_ROW,), jnp.float32),
            pltpu.SemaphoreType.DMA,
            pltpu.SemaphoreType.DMA,
            pltpu.SemaphoreType.DMA,
        ],
    )
    def sc_bins(pred_hbm, targ_hbm, batch_hbm, out_hbm,
                pred_v, targ_v, batch_v, bins_v, sem1, sem2, sem3):
        w = lax.axis_index("c") * 16 + lax.axis_index("s")
        base = w * _CH

        zeros16 = jnp.zeros((16,), jnp.float32)
        ones16 = jnp.ones((16,), jnp.float32)
        lane = lax.iota(jnp.int32, 16)

        def stage(n):
            dst = pl.ds(0, n)
            c1 = pltpu.async_copy(pred_hbm.at[pl.ds(base, n)], pred_v.at[dst], sem1)
            c2 = pltpu.async_copy(targ_hbm.at[pl.ds(base, n)], targ_v.at[dst], sem2)
            c3 = pltpu.async_copy(batch_hbm.at[pl.ds(base, n)], batch_v.at[dst], sem3)
            c1.wait()
            c2.wait()
            c3.wait()

        @pl.when(w != _NW - 1)
        def _stage_full():
            stage(_CH)

        @pl.when(w == _NW - 1)
        def _stage_tail():
            stage(_LAST_CH)

        for j in range(_ROW // 16):
            bins_v[pl.ds(j * 16, 16)] = zeros16

        def accumulate(n_vregs):
            lane_base = lane * n_vregs

            def body(i, carry):
                idx = lane_base + i
                p = plsc.load_gather(pred_v, [idx])
                t = plsc.load_gather(targ_v, [idx])
                b = plsc.load_gather(batch_v, [idx])
                v = jnp.abs(p * p - t * t)
                plsc.addupdate_scatter(bins_v, [b], v)
                plsc.addupdate_scatter(bins_v, [b + _BINS], ones16)
                return carry
            lax.fori_loop(0, n_vregs, body, 0, unroll=4)

        @pl.when(w != _NW - 1)
        def _accum_full():
            accumulate(_CH // 16)

        @pl.when(w == _NW - 1)
        def _accum_tail():
            accumulate(_LAST_CH // 16)
            last_vec = batch_v[pl.ds(_LAST_CH - 16, 16)]
            bins_v[pl.ds(2 * _BINS, 16)] = (
                zeros16 + last_vec[15].astype(jnp.float32))

        pltpu.sync_copy(bins_v, out_hbm.at[pl.ds(w * _ROW, _ROW)])

    return sc_bins


_sc_call = _make_sc_call()


def _tc_combine_body(x_ref, o_ref):
    x = x_ref[...]
    sums = jnp.sum(x[:, 0:_NSEG], axis=0, keepdims=True)
    cnts = jnp.sum(x[:, _BINS:_BINS + _NSEG], axis=0, keepdims=True)
    max_b = x[_NW - 1, 2 * _BINS].astype(jnp.int32)
    losses = sums / cnts
    seg = lax.broadcasted_iota(jnp.int32, (1, _NSEG), 1)
    valid = seg <= max_b
    total = jnp.sum(jnp.where(valid, losses, jnp.zeros_like(losses)))
    n_graphs = (max_b + 1).astype(jnp.float32)
    o_ref[...] = jnp.full((1, 1), (total / n_graphs) * 10000.0, jnp.float32)


_tc_combine = pl.pallas_call(
    _tc_combine_body,
    out_shape=jax.ShapeDtypeStruct((1, 1), jnp.float32),
)


@jax.jit
def kernel(pred, target, batch, x):
    del x
    rows = _sc_call(pred, target, batch).reshape(_NW, _ROW)
    return _tc_combine(rows)[0, 0]
